# Initial kernel scaffold; baseline (speedup 1.0000x reference)
import functools

import jax
import jax.numpy as jnp
from jax import lax
from jax.experimental import pallas as pl
from jax.experimental.pallas import tpu as pltpu
from jax.experimental.pallas import tpu_sc as plsc

N = 10000
E = 160000
H = 128
NC = 2
NS = 16
NW = NC * NS
CHUNK = 128
NCH = 40
E_PAD = NW * NCH * CHUNK
ROWS_PER_TILE = N // NS
ACC_ROWS = N + 16

_mesh = plsc.VectorSubcoreMesh(core_axis_name="c", subcore_axis_name="s")


def _sc_segment_sum(p, src2d, dst2d, r_half):
    @functools.partial(
        pl.kernel,
        out_type=jax.ShapeDtypeStruct((NC, N, H), jnp.float32),
        mesh=_mesh,
        scratch_types=[
            pltpu.VMEM_SHARED((ACC_ROWS, H), jnp.float32),
            pltpu.VMEM((NCH, CHUNK), jnp.int32),
            pltpu.VMEM((NCH, CHUNK), jnp.int32),
            pltpu.VMEM((CHUNK, H), jnp.float32),
            pltpu.SemaphoreType.DMA,
        ],
    )
    def run(p_hbm, src_hbm, dst_hbm, rh_hbm, out_hbm, acc, src_v, dst_v, rows_v, sem):
        c = lax.axis_index("c")
        s = lax.axis_index("s")
        wid = s * NC + c
        row0 = s * ROWS_PER_TILE
        pltpu.sync_copy(rh_hbm.at[pl.ds(row0, ROWS_PER_TILE)],
                        acc.at[pl.ds(row0, ROWS_PER_TILE)])
        pltpu.sync_copy(src_hbm.at[pl.ds(wid * NCH, NCH)], src_v)
        pltpu.sync_copy(dst_hbm.at[pl.ds(wid * NCH, NCH)], dst_v)
        plsc.subcore_barrier()

        def step(g, carry):
            pltpu.async_copy(p_hbm.at[src_v.at[g]], rows_v, sem).wait()
            pltpu.sync_copy(rows_v, acc.at[dst_v.at[g]], add=True)
            return carry

        lax.fori_loop(0, NCH, step, 0)
        plsc.subcore_barrier()
        pltpu.sync_copy(acc.at[pl.ds(row0, ROWS_PER_TILE)],
                        out_hbm.at[c, pl.ds(row0, ROWS_PER_TILE)])

    return run(p, src2d, dst2d, r_half)


_R = 2000


def _silu(v):
    return v * jax.nn.sigmoid(v)


def _tc_first_body(x_ref, w_ref, b_ref, p_ref, r_ref):
    t = jnp.dot(x_ref[...], w_ref[...], preferred_element_type=jnp.float32)
    p_ref[...] = t[:, :H]
    r_ref[...] = 0.5 * (t[:, H:] + b_ref[...])


def _tc_first(x, wcat, bl):
    din = x.shape[1]
    return pl.pallas_call(
        _tc_first_body,
        grid=(N // _R,),
        in_specs=[
            pl.BlockSpec((_R, din), lambda i: (i, 0)),
            pl.BlockSpec((din, 2 * H), lambda i: (0, 0)),
            pl.BlockSpec((1, H), lambda i: (0, 0)),
        ],
        out_specs=[
            pl.BlockSpec((_R, H), lambda i: (i, 0)),
            pl.BlockSpec((_R, H), lambda i: (i, 0)),
        ],
        out_shape=[
            jax.ShapeDtypeStruct((N, H), jnp.float32),
            jax.ShapeDtypeStruct((N, H), jnp.float32),
        ],
    )(x, wcat, bl.reshape(1, H))


def _tc_mid_body(a_ref, w_ref, b_ref, p_ref, r_ref):
    h = _silu(a_ref[0] + a_ref[1])
    t = jnp.dot(h, w_ref[...], preferred_element_type=jnp.float32)
    p_ref[...] = t[:, :H]
    r_ref[...] = 0.5 * (t[:, H:] + b_ref[...])


def _tc_mid(a, wcat, bl):
    return pl.pallas_call(
        _tc_mid_body,
        grid=(N // _R,),
        in_specs=[
            pl.BlockSpec((NC, _R, H), lambda i: (0, i, 0)),
            pl.BlockSpec((H, 2 * H), lambda i: (0, 0)),
            pl.BlockSpec((1, H), lambda i: (0, 0)),
        ],
        out_specs=[
            pl.BlockSpec((_R, H), lambda i: (i, 0)),
            pl.BlockSpec((_R, H), lambda i: (i, 0)),
        ],
        out_shape=[
            jax.ShapeDtypeStruct((N, H), jnp.float32),
            jax.ShapeDtypeStruct((N, H), jnp.float32),
        ],
    )(a, wcat, bl.reshape(1, H))


def _tc_final_body(a_ref, w1_ref, b1_ref, w2_ref, b2_ref, w3_ref, b3_ref, o_ref):
    h = _silu(a_ref[0] + a_ref[1])
    y = _silu(jnp.dot(h, w1_ref[...], preferred_element_type=jnp.float32)
              + b1_ref[...])
    y = _silu(jnp.dot(y, w2_ref[...], preferred_element_type=jnp.float32)
              + b2_ref[...])
    o_ref[...] = (jnp.dot(y, w3_ref[...], preferred_element_type=jnp.float32)
                  + b3_ref[...])


def _tc_final(a, w1, b1, w2, b2, w3, b3):
    return pl.pallas_call(
        _tc_final_body,
        grid=(N // _R,),
        in_specs=[
            pl.BlockSpec((NC, _R, H), lambda i: (0, i, 0)),
            pl.BlockSpec((H, 2 * H), lambda i: (0, 0)),
            pl.BlockSpec((1, 2 * H), lambda i: (0, 0)),
            pl.BlockSpec((2 * H, 2 * H), lambda i: (0, 0)),
            pl.BlockSpec((1, 2 * H), lambda i: (0, 0)),
            pl.BlockSpec((2 * H, 1), lambda i: (0, 0)),
            pl.BlockSpec((1, 1), lambda i: (0, 0)),
        ],
        out_specs=pl.BlockSpec((_R, 1), lambda i: (i, 0)),
        out_shape=jax.ShapeDtypeStruct((N, 1), jnp.float32),
    )(a, w1, b1.reshape(1, 2 * H), w2, b2.reshape(1, 2 * H), w3,
      b3.reshape(1, 1))


def kernel(x, edge_index, Wl0, bl0, Wr0, Wl1, bl1, Wr1, Wl2, bl2, Wr2,
           Wl3, bl3, Wr3, W1, b1, W2, b2, W3, b3):
    src = edge_index[0]
    dst = edge_index[1]
    pad = E_PAD - E
    src2d = jnp.concatenate(
        [src, jnp.zeros((pad,), jnp.int32)]).reshape(E_PAD // CHUNK, CHUNK)
    dst2d = jnp.concatenate(
        [dst, jnp.full((pad,), N, jnp.int32)]).reshape(E_PAD // CHUNK, CHUNK)

    wcats = [jnp.concatenate([wl, wr], axis=1)
             for wl, wr in ((Wl0, Wr0), (Wl1, Wr1), (Wl2, Wr2), (Wl3, Wr3))]
    bls = (bl0, bl1, bl2, bl3)

    p, r_half = _tc_first(x, wcats[0], bls[0])
    a = _sc_segment_sum(p, src2d, dst2d, r_half)
    for i in (1, 2, 3):
        p, r_half = _tc_mid(a, wcats[i], bls[i])
        a = _sc_segment_sum(p, src2d, dst2d, r_half)
    return _tc_final(a, W1, b1, W2, b2, W3, b3)


# speedup vs baseline: 3.1557x; 3.1557x over previous
import functools

import jax
import jax.numpy as jnp
from jax import lax
from jax.experimental import pallas as pl
from jax.experimental.pallas import tpu as pltpu
from jax.experimental.pallas import tpu_sc as plsc

N = 10000
E = 160000
H = 128
NC = 2
NS = 16
NW = NC * NS
CHUNK = 128
NCH = 40
E_PAD = NW * NCH * CHUNK
ROWS_PER_TILE = 624
ACC_ROWS = N + 16

_mesh = plsc.VectorSubcoreMesh(core_axis_name="c", subcore_axis_name="s")


def _sc_segment_sum(p, src2d, dst2d, r_half):
    @functools.partial(
        pl.kernel,
        out_type=jax.ShapeDtypeStruct((NC, N, H), jnp.float32),
        mesh=_mesh,
        scratch_types=[
            pltpu.VMEM_SHARED((ACC_ROWS, H), jnp.float32),
            pltpu.VMEM((NCH, CHUNK), jnp.int32),
            pltpu.VMEM((NCH, CHUNK), jnp.int32),
            pltpu.VMEM((CHUNK, H), jnp.float32),
            pltpu.SemaphoreType.DMA,
        ],
    )
    def run(p_hbm, src_hbm, dst_hbm, rh_hbm, out_hbm, acc, src_v, dst_v, rows_v, sem):
        c = lax.axis_index("c")
        s = lax.axis_index("s")
        wid = s * NC + c
        row0 = s * ROWS_PER_TILE
        pltpu.sync_copy(rh_hbm.at[pl.ds(row0, ROWS_PER_TILE)],
                        acc.at[pl.ds(row0, ROWS_PER_TILE)])

        @pl.when(s == NS - 1)
        def _prefill_tail():
            pltpu.sync_copy(rh_hbm.at[pl.ds(NS * ROWS_PER_TILE, N - NS * ROWS_PER_TILE)],
                            acc.at[pl.ds(NS * ROWS_PER_TILE, N - NS * ROWS_PER_TILE)])

        pltpu.sync_copy(src_hbm.at[pl.ds(wid * NCH, NCH)], src_v)
        pltpu.sync_copy(dst_hbm.at[pl.ds(wid * NCH, NCH)], dst_v)
        plsc.subcore_barrier()

        def step(g, carry):
            pltpu.async_copy(p_hbm.at[src_v.at[g]], rows_v, sem).wait()
            pltpu.sync_copy(rows_v, acc.at[dst_v.at[g]], add=True)
            return carry

        lax.fori_loop(0, NCH, step, 0)
        plsc.subcore_barrier()
        pltpu.sync_copy(acc.at[pl.ds(row0, ROWS_PER_TILE)],
                        out_hbm.at[c, pl.ds(row0, ROWS_PER_TILE)])

        @pl.when(s == NS - 1)
        def _writeback_tail():
            pltpu.sync_copy(acc.at[pl.ds(NS * ROWS_PER_TILE, N - NS * ROWS_PER_TILE)],
                            out_hbm.at[c, pl.ds(NS * ROWS_PER_TILE, N - NS * ROWS_PER_TILE)])

    return run(p, src2d, dst2d, r_half)


_R = 2000


def _silu(v):
    return v * jax.nn.sigmoid(v)


def _tc_first_body(x_ref, w_ref, b_ref, p_ref, r_ref):
    t = jnp.dot(x_ref[...], w_ref[...], preferred_element_type=jnp.float32)
    p_ref[...] = t[:, :H]
    r_ref[...] = 0.5 * (t[:, H:] + b_ref[...])


def _tc_first(x, wcat, bl):
    din = x.shape[1]
    return pl.pallas_call(
        _tc_first_body,
        grid=(N // _R,),
        in_specs=[
            pl.BlockSpec((_R, din), lambda i: (i, 0)),
            pl.BlockSpec((din, 2 * H), lambda i: (0, 0)),
            pl.BlockSpec((1, H), lambda i: (0, 0)),
        ],
        out_specs=[
            pl.BlockSpec((_R, H), lambda i: (i, 0)),
            pl.BlockSpec((_R, H), lambda i: (i, 0)),
        ],
        out_shape=[
            jax.ShapeDtypeStruct((N, H), jnp.float32),
            jax.ShapeDtypeStruct((N, H), jnp.float32),
        ],
    )(x, wcat, bl.reshape(1, H))


def _tc_mid_body(a_ref, w_ref, b_ref, p_ref, r_ref):
    h = _silu(a_ref[0] + a_ref[1])
    t = jnp.dot(h, w_ref[...], preferred_element_type=jnp.float32)
    p_ref[...] = t[:, :H]
    r_ref[...] = 0.5 * (t[:, H:] + b_ref[...])


def _tc_mid(a, wcat, bl):
    return pl.pallas_call(
        _tc_mid_body,
        grid=(N // _R,),
        in_specs=[
            pl.BlockSpec((NC, _R, H), lambda i: (0, i, 0)),
            pl.BlockSpec((H, 2 * H), lambda i: (0, 0)),
            pl.BlockSpec((1, H), lambda i: (0, 0)),
        ],
        out_specs=[
            pl.BlockSpec((_R, H), lambda i: (i, 0)),
            pl.BlockSpec((_R, H), lambda i: (i, 0)),
        ],
        out_shape=[
            jax.ShapeDtypeStruct((N, H), jnp.float32),
            jax.ShapeDtypeStruct((N, H), jnp.float32),
        ],
    )(a, wcat, bl.reshape(1, H))


def _tc_final_body(a_ref, w1_ref, b1_ref, w2_ref, b2_ref, w3_ref, b3_ref, o_ref):
    h = _silu(a_ref[0] + a_ref[1])
    y = _silu(jnp.dot(h, w1_ref[...], preferred_element_type=jnp.float32)
              + b1_ref[...])
    y = _silu(jnp.dot(y, w2_ref[...], preferred_element_type=jnp.float32)
              + b2_ref[...])
    o_ref[...] = (jnp.dot(y, w3_ref[...], preferred_element_type=jnp.float32)
                  + b3_ref[...])


def _tc_final(a, w1, b1, w2, b2, w3, b3):
    return pl.pallas_call(
        _tc_final_body,
        grid=(N // _R,),
        in_specs=[
            pl.BlockSpec((NC, _R, H), lambda i: (0, i, 0)),
            pl.BlockSpec((H, 2 * H), lambda i: (0, 0)),
            pl.BlockSpec((1, 2 * H), lambda i: (0, 0)),
            pl.BlockSpec((2 * H, 2 * H), lambda i: (0, 0)),
            pl.BlockSpec((1, 2 * H), lambda i: (0, 0)),
            pl.BlockSpec((2 * H, 1), lambda i: (0, 0)),
            pl.BlockSpec((1, 1), lambda i: (0, 0)),
        ],
        out_specs=pl.BlockSpec((_R, 1), lambda i: (i, 0)),
        out_shape=jax.ShapeDtypeStruct((N, 1), jnp.float32),
    )(a, w1, b1.reshape(1, 2 * H), w2, b2.reshape(1, 2 * H), w3,
      b3.reshape(1, 1))


def kernel(x, edge_index, Wl0, bl0, Wr0, Wl1, bl1, Wr1, Wl2, bl2, Wr2,
           Wl3, bl3, Wr3, W1, b1, W2, b2, W3, b3):
    src = edge_index[0]
    dst = edge_index[1]
    pad = E_PAD - E
    src2d = jnp.concatenate(
        [src, jnp.zeros((pad,), jnp.int32)]).reshape(E_PAD // CHUNK, CHUNK)
    dst2d = jnp.concatenate(
        [dst, jnp.full((pad,), N, jnp.int32)]).reshape(E_PAD // CHUNK, CHUNK)

    wcats = [jnp.concatenate([wl, wr], axis=1)
             for wl, wr in ((Wl0, Wr0), (Wl1, Wr1), (Wl2, Wr2), (Wl3, Wr3))]
    bls = (bl0, bl1, bl2, bl3)

    p, r_half = _tc_first(x, wcats[0], bls[0])
    a = _sc_segment_sum(p, src2d, dst2d, r_half)
    for i in (1, 2, 3):
        p, r_half = _tc_mid(a, wcats[i], bls[i])
        a = _sc_segment_sum(p, src2d, dst2d, r_half)
    return _tc_final(a, W1, b1, W2, b2, W3, b3)
